# Initial kernel scaffold; baseline (speedup 1.0000x reference)
import functools

import jax
import jax.numpy as jnp
from jax import lax
from jax.experimental import pallas as pl
from jax.experimental.pallas import tpu as pltpu
from jax.experimental.pallas import tpu_sc as plsc

_F = 26
_D = 64
_B = 4096
_N = _B * _F

_NC, _NS = 2, 16
_NW = _NC * _NS
_PER_W = _N // _NW
_CHUNK = 128
_NCHUNK = _PER_W // _CHUNK

_mesh = plsc.VectorSubcoreMesh(core_axis_name="c", subcore_axis_name="s")


@functools.partial(
    pl.kernel,
    mesh=_mesh,
    out_type=jax.ShapeDtypeStruct((_N, _D), jnp.float32),
    scratch_types=[
        pltpu.VMEM((_NCHUNK, _CHUNK), jnp.int32),
        pltpu.VMEM((2, _CHUNK, _D), jnp.float32),
        pltpu.SemaphoreType.DMA,
    ],
)
def _gather(idx_hbm, table_hbm, out_hbm, idx_v, rows_v, gsem):
    wid = lax.axis_index("s") * _NC + lax.axis_index("c")
    base = wid * _PER_W
    pltpu.sync_copy(idx_hbm.at[wid], idx_v)
    pending = [None, None]
    for j in range(_NCHUNK):
        pending[j % 2] = pltpu.async_copy(
            table_hbm.at[idx_v.at[j]], rows_v.at[j % 2], gsem)
        if j > 0:
            pending[(j - 1) % 2].wait()
            pltpu.sync_copy(
                rows_v.at[(j - 1) % 2],
                out_hbm.at[pl.ds(base + (j - 1) * _CHUNK, _CHUNK)])
    pending[(_NCHUNK - 1) % 2].wait()
    pltpu.sync_copy(
        rows_v.at[(_NCHUNK - 1) % 2],
        out_hbm.at[pl.ds(base + (_NCHUNK - 1) * _CHUNK, _CHUNK)])


def kernel(X, table):
    idx = X.reshape(_NW, _NCHUNK, _CHUNK)
    flat = _gather(idx, table)
    out = flat.reshape(_B, _F * _D)
    return (out, out)


# speedup vs baseline: 1.6526x; 1.6526x over previous
import functools

import jax
import jax.numpy as jnp
from jax import lax
from jax.experimental import pallas as pl
from jax.experimental.pallas import tpu as pltpu
from jax.experimental.pallas import tpu_sc as plsc

_F = 26
_D = 64
_B = 4096
_N = _B * _F

_NC, _NS = 2, 16
_NW = _NC * _NS
_PER_W = _N // _NW
_CHUNK = 128
_NCHUNK = _PER_W // _CHUNK

_mesh = plsc.VectorSubcoreMesh(core_axis_name="c", subcore_axis_name="s")


@functools.partial(
    pl.kernel,
    mesh=_mesh,
    out_type=jax.ShapeDtypeStruct((_N, _D), jnp.float32),
    scratch_types=[
        pltpu.VMEM((_NCHUNK, _CHUNK), jnp.int32),
        pltpu.VMEM((2, _CHUNK, _D), jnp.float32),
        pltpu.SemaphoreType.DMA,
    ],
    compiler_params=pltpu.CompilerParams(use_tc_tiling_on_sc=False),
)
def _gather(idx_hbm, table_hbm, out_hbm, idx_v, rows_v, gsem):
    wid = lax.axis_index("s") * _NC + lax.axis_index("c")
    base = wid * _PER_W
    pltpu.sync_copy(idx_hbm.at[wid], idx_v)
    pending = [None, None]
    for j in range(_NCHUNK):
        pending[j % 2] = pltpu.async_copy(
            table_hbm.at[idx_v.at[j]], rows_v.at[j % 2], gsem)
        if j > 0:
            pending[(j - 1) % 2].wait()
            pltpu.sync_copy(
                rows_v.at[(j - 1) % 2],
                out_hbm.at[pl.ds(base + (j - 1) * _CHUNK, _CHUNK)])
    pending[(_NCHUNK - 1) % 2].wait()
    pltpu.sync_copy(
        rows_v.at[(_NCHUNK - 1) % 2],
        out_hbm.at[pl.ds(base + (_NCHUNK - 1) * _CHUNK, _CHUNK)])


def kernel(X, table):
    idx = X.reshape(_NW, _NCHUNK, _CHUNK)
    flat = _gather(idx, table)
    out = flat.reshape(_B, _F * _D)
    return (out, out)


# speedup vs baseline: 1.6960x; 1.0263x over previous
import functools

import jax
import jax.numpy as jnp
from jax import lax
from jax.experimental import pallas as pl
from jax.experimental.pallas import tpu as pltpu
from jax.experimental.pallas import tpu_sc as plsc

_F = 26
_D = 64
_B = 4096
_N = _B * _F

_NC, _NS = 2, 16
_NW = _NC * _NS
_PER_W = _N // _NW
_CHUNK = 128
_NCHUNK = _PER_W // _CHUNK
_NBUF = 4

_mesh = plsc.VectorSubcoreMesh(core_axis_name="c", subcore_axis_name="s")


@functools.partial(
    pl.kernel,
    mesh=_mesh,
    out_type=jax.ShapeDtypeStruct((_N, _D), jnp.float32),
    scratch_types=[
        pltpu.VMEM((_NCHUNK, _CHUNK), jnp.int32),
        pltpu.VMEM((_NBUF, _CHUNK, _D), jnp.float32),
    ]
    + [pltpu.SemaphoreType.DMA] * (2 * _NBUF),
    compiler_params=pltpu.CompilerParams(use_tc_tiling_on_sc=False),
)
def _gather(idx_hbm, table_hbm, out_hbm, idx_v, rows_v, *sems):
    gsems, osems = sems[:_NBUF], sems[_NBUF:]
    wid = lax.axis_index("s") * _NC + lax.axis_index("c")
    base = wid * _PER_W
    pltpu.sync_copy(idx_hbm.at[wid], idx_v)
    pend_g = [None] * _NBUF
    pend_o = [None] * _NBUF

    def _writeback(j):
        b = j % _NBUF
        pend_g[b].wait()
        pend_o[b] = pltpu.async_copy(
            rows_v.at[b], out_hbm.at[pl.ds(base + j * _CHUNK, _CHUNK)],
            osems[b])

    for j in range(_NCHUNK):
        b = j % _NBUF
        if pend_o[b] is not None:
            pend_o[b].wait()
        pend_g[b] = pltpu.async_copy(
            table_hbm.at[idx_v.at[j]], rows_v.at[b], gsems[b])
        if j >= _NBUF - 1:
            _writeback(j - (_NBUF - 1))
    for j in range(max(0, _NCHUNK - (_NBUF - 1)), _NCHUNK):
        _writeback(j)
    for b in range(_NBUF):
        if pend_o[b] is not None:
            pend_o[b].wait()


def kernel(X, table):
    idx = X.reshape(_NW, _NCHUNK, _CHUNK)
    flat = _gather(idx, table)
    out = flat.reshape(_B, _F * _D)
    return (out, out)
